# baseline (speedup 1.0000x reference)
import jax
import jax.numpy as jnp
from jax import lax
from jax.experimental import pallas as pl
from jax.experimental.pallas import tpu as pltpu
from jax.experimental.pallas import tpu_sc as plsc

_N, _E, _D, _L = 10000, 160000, 128, 4
_NW = 32
_EPT = 5120
_EP = _NW * _EPT
_BLK = 16
_SB = 128
_SBB = _SB // _BLK
_CHK = 16
_NB = _EPT // _BLK
_NA = 10240
_NSUB = 16
_STRIPE = _NA // _NSUB
_RB = 80
_EB = 128
_EPS_BN = 1e-5
_EPS_DEN = 1e-6


def _node_mm_body(h_ref, w1, w2, w3, w5, w6, b1, b2, b3, b5, b6,
                  a1h, a2h, a3h, b2h, b3h):
    x = h_ref[...]

    def mm(w, b):
        return jnp.dot(x, w[...], preferred_element_type=jnp.float32) + b[...]

    a1h[...] = mm(w1, b1)
    a2h[...] = mm(w2, b2)
    a3h[...] = mm(w3, b3)
    b2h[...] = mm(w5, b5)
    b3h[...] = mm(w6, b6)


def _node_mm(h, ws, bs):
    row = pl.BlockSpec((_RB, _D), lambda i: (i, 0))
    wfull = pl.BlockSpec((_D, _D), lambda i: (0, 0))
    wbias = pl.BlockSpec((1, _D), lambda i: (0, 0))
    return pl.pallas_call(
        _node_mm_body,
        grid=(_N // _RB,),
        in_specs=[row] + [wfull] * 5 + [wbias] * 5,
        out_specs=[row] * 5,
        out_shape=[jax.ShapeDtypeStruct((_N, _D), jnp.float32)]
        + [jax.ShapeDtypeStruct((_NA, _D), jnp.float32)] * 4,
    )(h, *ws, *[b.reshape(1, _D) for b in bs])


def _edge_mm_body(e_ref, w, b, out):
    out[...] = (jnp.dot(e_ref[...], w[...], preferred_element_type=jnp.float32)
                + b[...])


def _edge_mm(e, w, b):
    nbe = _E // _EB - 1
    return pl.pallas_call(
        _edge_mm_body,
        grid=(_EP // _EB,),
        in_specs=[
            pl.BlockSpec((_EB, _D), lambda i: (jnp.minimum(i, nbe), 0)),
            pl.BlockSpec((_D, _D), lambda i: (0, 0)),
            pl.BlockSpec((1, _D), lambda i: (0, 0)),
        ],
        out_specs=pl.BlockSpec((_EB, _D), lambda i: (i, 0)),
        out_shape=jax.ShapeDtypeStruct((_EP, _D), jnp.float32),
    )(e, w, b.reshape(1, _D))


_MESH = plsc.VectorSubcoreMesh(core_axis_name="c", subcore_axis_name="s",
                               num_cores=2, num_subcores=_NSUB)
_SC_PARAMS = pltpu.CompilerParams(use_tc_tiling_on_sc=False)


def _zero_buf(zbuf):
    def zrow(i, carry):
        r = i // (_D // 16)
        col = (i % (_D // 16)) * 16
        zbuf[r, pl.ds(col, 16)] = jnp.zeros((16,), jnp.float32)
        return carry

    lax.fori_loop(0, (_SB * _D) // 16, zrow, 0)


def _zero_acc_stripe(acc, zbuf, sbase):
    for j in range(_STRIPE // _SB):
        pltpu.sync_copy(zbuf, acc.at[pl.ds(sbase + j * _SB, _SB)])


def _sigmoid16(g):
    return 1.0 / (1.0 + jnp.exp(-g))


def _make_sc_msg_pass():
    out_type = [jax.ShapeDtypeStruct((_EP, _D), jnp.float32),
                jax.ShapeDtypeStruct((2, _NA, _D), jnp.float32)]

    def buf():
        return pltpu.VMEM((_BLK, _D), jnp.float32)

    scratch = (
        [pltpu.VMEM((_NB // _SBB, _SB), jnp.int32)] * 2
        + [buf()] * 10
        + [pltpu.VMEM((_SB, _D), jnp.float32)]
        + [pltpu.VMEM_SHARED((_NA, _D), jnp.float32)]
        + [pltpu.SemaphoreType.DMA] * 4
    )

    def body(iu_hbm, iv_hbm, t2, t3, ta, b1e, gate, ph, iu_v, iv_v, *rest):
        sets = (rest[0:5], rest[5:10])
        mo_big = rest[10]
        acc = rest[11]
        gsems = (rest[12], rest[13])
        ssems = (rest[14], rest[15])

        c = lax.axis_index("c")
        s = lax.axis_index("s")
        wid = c * _NSUB + s
        base_e = wid * _EPT
        sbase = s * _STRIPE

        pltpu.sync_copy(iu_hbm.at[wid], iu_v)
        pltpu.sync_copy(iv_hbm.at[wid], iv_v)
        _zero_buf(mo_big)
        _zero_acc_stripe(acc, mo_big, sbase)
        plsc.subcore_barrier()

        def islice(slab, k):
            return slab.at[k // _SBB, pl.ds((k % _SBB) * _BLK, _BLK)]

        def issue_g(k, p):
            bu, bv, ba, b1 = sets[p][0:4]
            sem = gsems[p]
            return (pltpu.async_copy(t2.at[islice(iu_v, k)], bu, sem),
                    pltpu.async_copy(t3.at[islice(iv_v, k)], bv, sem),
                    pltpu.async_copy(ta.at[islice(iu_v, k)], ba, sem),
                    pltpu.async_copy(b1e.at[pl.ds(base_e + k * _BLK, _BLK)],
                                     b1, sem))

        def issue_gate(k, p):
            go = sets[p][4]
            return (pltpu.async_copy(go,
                                     gate.at[pl.ds(base_e + k * _BLK, _BLK)],
                                     ssems[p]),)

        def compute(p, rbase):
            bu, bv, ba, b1, go = sets[p]
            sls = [pl.ds(i * 16, 16) for i in range(_D // 16)]

            def rows(r2, cr):
                rr = [2 * r2, 2 * r2 + 1]
                pos = [(r, sl) for r in rr for sl in sls]
                gs = [b1[r, sl] + bu[r, sl] + bv[r, sl] for r, sl in pos]
                es = [jnp.exp(-g) for g in gs]
                sg = [1.0 / (1.0 + e2) for e2 in es]
                for i, (r, sl) in enumerate(pos):
                    go[r, sl] = gs[i]
                for i, (r, sl) in enumerate(pos):
                    mo_big[rbase + r, sl] = sg[i] * ba[r, sl]
                return cr

            lax.fori_loop(0, _BLK // 2, rows, 0)

        def chunk(j, carry):
            base = j * _CHK
            gd = {0: issue_g(base, 0), 1: issue_g(base + 1, 1)}
            sd = {}
            for bi in range(_CHK):
                p = bi % 2
                for d in gd.pop(bi):
                    d.wait()
                if bi >= 2:
                    for d in sd.pop(bi - 2):
                        d.wait()
                compute(p, (bi % _SBB) * _BLK)
                sd[bi] = issue_gate(base + bi, p)
                if bi % _SBB == _SBB - 1:
                    pltpu.sync_copy(
                        mo_big, acc.at[iv_v.at[(base + bi) // _SBB]],
                        add=True)
                if bi + 2 < _CHK:
                    gd[bi + 2] = issue_g(base + bi + 2, p)
            for d in sd.pop(_CHK - 2):
                d.wait()
            for d in sd.pop(_CHK - 1):
                d.wait()
            return carry

        lax.fori_loop(0, _NB // _CHK, chunk, 0)

        plsc.subcore_barrier()
        pltpu.sync_copy(acc.at[pl.ds(sbase, _STRIPE)],
                        ph.at[c, pl.ds(sbase, _STRIPE)])

    return pl.kernel(body, out_type=out_type, mesh=_MESH,
                     scratch_types=scratch, compiler_params=_SC_PARAMS)


def _make_sc_sig_pass():
    out_type = [jax.ShapeDtypeStruct((2, _NA, _D), jnp.float32)] * 2

    def buf():
        return pltpu.VMEM((_BLK, _D), jnp.float32)

    scratch = (
        [pltpu.VMEM((_NB // _SBB, _SB), jnp.int32)]
        + [buf()] * 2
        + [pltpu.VMEM((_SB, _D), jnp.float32)]
        + [pltpu.VMEM_SHARED((_NA, _D), jnp.float32)]
        + [pltpu.SemaphoreType.DMA] * 2
    )

    def body(dst_hbm, src_hbm, gate_f, gate_b, ps_f, ps_b, iv_v, *rest):
        sets = (rest[0], rest[1])
        so_big = rest[2]
        acc = rest[3]
        gsems = (rest[4], rest[5])

        c = lax.axis_index("c")
        s = lax.axis_index("s")
        wid = c * _NSUB + s
        base_e = wid * _EPT
        sbase = s * _STRIPE

        for idx_hbm, gate, ps in ((dst_hbm, gate_f, ps_f),
                                  (src_hbm, gate_b, ps_b)):
            pltpu.sync_copy(idx_hbm.at[wid], iv_v)
            _zero_buf(so_big)
            _zero_acc_stripe(acc, so_big, sbase)
            plsc.subcore_barrier()

            def issue_g(k, p):
                return (pltpu.async_copy(
                    gate.at[pl.ds(base_e + k * _BLK, _BLK)],
                    sets[p], gsems[p]),)

            def compute(p, rbase):
                bg = sets[p]
                sls = [pl.ds(i * 16, 16) for i in range(_D // 16)]

                def rows(r2, cr):
                    rr = [2 * r2, 2 * r2 + 1]
                    pos = [(r, sl) for r in rr for sl in sls]
                    es = [jnp.exp(-bg[r, sl]) for r, sl in pos]
                    sg = [1.0 / (1.0 + e2) for e2 in es]
                    for i, (r, sl) in enumerate(pos):
                        so_big[rbase + r, sl] = sg[i]
                    return cr

                lax.fori_loop(0, _BLK // 2, rows, 0)

            def chunk(j, carry):
                base = j * _CHK
                gd = {0: issue_g(base, 0), 1: issue_g(base + 1, 1)}
                for bi in range(_CHK):
                    p = bi % 2
                    for d in gd.pop(bi):
                        d.wait()
                    compute(p, (bi % _SBB) * _BLK)
                    if bi % _SBB == _SBB - 1:
                        pltpu.sync_copy(
                            so_big, acc.at[iv_v.at[(base + bi) // _SBB]],
                            add=True)
                    if bi + 2 < _CHK:
                        gd[bi + 2] = issue_g(base + bi + 2, p)
                return carry

            lax.fori_loop(0, _NB // _CHK, chunk, 0)

            plsc.subcore_barrier()
            pltpu.sync_copy(acc.at[pl.ds(sbase, _STRIPE)],
                            ps.at[c, pl.ds(sbase, _STRIPE)])
            plsc.subcore_barrier()

    return pl.kernel(body, out_type=out_type, mesh=_MESH,
                     scratch_types=scratch, compiler_params=_SC_PARAMS)


_sc_msg_pass = _make_sc_msg_pass()
_sc_sig_pass = _make_sc_sig_pass()


def _hfin_body(hin, a1h, fh, fs, bh, bs, g, b, out, stats):
    p = pl.program_id(0)
    i = pl.program_id(1)

    def comb(x):
        return jnp.sum(x[...], axis=0)

    hn = (a1h[...] + comb(fh) / (comb(fs) + _EPS_DEN)
          + comb(bh) / (comb(bs) + _EPS_DEN))

    @pl.when(jnp.logical_and(p == 0, i == 0))
    def _():
        stats[...] = jnp.zeros_like(stats)

    @pl.when(p == 0)
    def _():
        stats[0:1, :] = stats[0:1, :] + jnp.sum(hn, axis=0, keepdims=True)
        stats[1:2, :] = stats[1:2, :] + jnp.sum(hn * hn, axis=0, keepdims=True)
        out[...] = hn

    @pl.when(p == 1)
    def _():
        mu = stats[0:1, :] / _N
        var = stats[1:2, :] / _N - mu * mu
        xb = (hn - mu) * lax.rsqrt(var + _EPS_BN) * g[...] + b[...]
        out[...] = hin[...] + jnp.maximum(xb, 0.0)


def _h_finalize(hin, a1h, fh, fs, bh, bs, g, b):
    row = pl.BlockSpec((_RB, _D), lambda p, i: (i, 0))
    part = pl.BlockSpec((2, _RB, _D), lambda p, i: (0, i, 0))
    vec = pl.BlockSpec((1, _D), lambda p, i: (0, 0))
    return pl.pallas_call(
        _hfin_body,
        grid=(2, _N // _RB),
        in_specs=[row, row] + [part] * 4 + [vec, vec],
        out_specs=row,
        out_shape=jax.ShapeDtypeStruct((_N, _D), jnp.float32),
        scratch_shapes=[pltpu.VMEM((8, _D), jnp.float32)],
    )(hin, a1h, fh, fs, bh, bs, g.reshape(1, _D), b.reshape(1, _D))


def _efin_body(ein, gref, g, b, out, stats):
    p = pl.program_id(0)
    i = pl.program_id(1)
    ge = gref[...]

    @pl.when(jnp.logical_and(p == 0, i == 0))
    def _():
        stats[...] = jnp.zeros_like(stats)

    @pl.when(p == 0)
    def _():
        stats[0:1, :] = stats[0:1, :] + jnp.sum(ge, axis=0, keepdims=True)
        stats[1:2, :] = stats[1:2, :] + jnp.sum(ge * ge, axis=0, keepdims=True)
        out[...] = ge

    @pl.when(p == 1)
    def _():
        mu = stats[0:1, :] / _E
        var = stats[1:2, :] / _E - mu * mu
        xb = (ge - mu) * lax.rsqrt(var + _EPS_BN) * g[...] + b[...]
        out[...] = ein[...] + jnp.maximum(xb, 0.0)


def _e_finalize(ein, gate, g, b):
    row = pl.BlockSpec((_EB, _D), lambda p, i: (i, 0))
    vec = pl.BlockSpec((1, _D), lambda p, i: (0, 0))
    return pl.pallas_call(
        _efin_body,
        grid=(2, _E // _EB),
        in_specs=[row, row, vec, vec],
        out_specs=row,
        out_shape=jax.ShapeDtypeStruct((_E, _D), jnp.float32),
        scratch_shapes=[pltpu.VMEM((8, _D), jnp.float32)],
    )(ein, gate, g.reshape(1, _D), b.reshape(1, _D))


def _layer(h, e, src3, dst3, A1w, A1b, A2w, A2b, A3w, A3b,
           B1w, B1b, B2w, B2b, B3w, B3b, gh, bh, ge, be):
    a1h, a2h, a3h, b2h, b3h = _node_mm(
        h, (A1w, A2w, A3w, B2w, B3w), (A1b, A2b, A3b, B2b, B3b))
    b1e = _edge_mm(e, B1w, B1b)

    gate_f, fh = _sc_msg_pass(src3, dst3, b2h, b3h, a2h, b1e)
    gate_b, bhp = _sc_msg_pass(dst3, src3, b2h, b3h, a3h, b1e)
    fs, bs = _sc_sig_pass(dst3, src3, gate_f, gate_b)

    h_out = _h_finalize(h, a1h, fh, fs, bhp, bs, gh, bh)
    e_out = _e_finalize(e, gate_f, ge, be)
    return h_out, e_out


def kernel(h, e, edge_index, A1_w, A1_b, A2_w, A2_b, A3_w, A3_b,
           B1_w, B1_b, B2_w, B2_b, B3_w, B3_b, bn_h_g, bn_h_b,
           bn_e_g, bn_e_b):
    pad = jnp.full((_EP - _E,), _N, dtype=jnp.int32)
    src3 = jnp.concatenate([edge_index[0], pad]).reshape(_NW, _NB // _SBB, _SB)
    dst3 = jnp.concatenate([edge_index[1], pad]).reshape(_NW, _NB // _SBB, _SB)
    for i in range(_L):
        h, e = _layer(h, e, src3, dst3,
                      A1_w[i], A1_b[i], A2_w[i], A2_b[i], A3_w[i], A3_b[i],
                      B1_w[i], B1_b[i], B2_w[i], B2_b[i], B3_w[i], B3_b[i],
                      bn_h_g[i], bn_h_b[i], bn_e_g[i], bn_e_b[i])
    return (h, e)


# speedup vs baseline: 2.0585x; 2.0585x over previous
import jax
import jax.numpy as jnp
from jax import lax
from jax.experimental import pallas as pl
from jax.experimental.pallas import tpu as pltpu
from jax.experimental.pallas import tpu_sc as plsc

_N, _E, _D, _L = 10000, 160000, 128, 4
_NW = 32
_EPT = 5120
_EP = _NW * _EPT
_BLK = 16
_SB = 128
_SBB = _SB // _BLK
_CHK = 16
_NB = _EPT // _BLK
_NA = 10240
_NSUB = 16
_STRIPE = _NA // _NSUB
_RB = 1000
_EB = 640
_EPS_BN = 1e-5
_EPS_DEN = 1e-6


def _node_mm_body(h_ref, w1, w2, w3, w5, w6, b1, b2, b3, b5, b6,
                  a1h, a2h, a3h, b2h, b3h):
    x = h_ref[...]

    def mm(w, b):
        return jnp.dot(x, w[...], preferred_element_type=jnp.float32) + b[...]

    a1h[...] = mm(w1, b1)
    a2h[...] = mm(w2, b2)
    a3h[...] = mm(w3, b3)
    b2h[...] = mm(w5, b5)
    b3h[...] = mm(w6, b6)


def _node_mm(h, ws, bs):
    row = pl.BlockSpec((_RB, _D), lambda i: (i, 0))
    wfull = pl.BlockSpec((_D, _D), lambda i: (0, 0))
    wbias = pl.BlockSpec((1, _D), lambda i: (0, 0))
    return pl.pallas_call(
        _node_mm_body,
        grid=(_N // _RB,),
        in_specs=[row] + [wfull] * 5 + [wbias] * 5,
        out_specs=[row] * 5,
        out_shape=[jax.ShapeDtypeStruct((_N, _D), jnp.float32)]
        + [jax.ShapeDtypeStruct((_NA, _D), jnp.float32)] * 4,
    )(h, *ws, *[b.reshape(1, _D) for b in bs])


def _edge_mm_body(e_ref, w, b, out):
    out[...] = (jnp.dot(e_ref[...], w[...], preferred_element_type=jnp.float32)
                + b[...])


def _edge_mm(e, w, b):
    nbe = _E // _EB - 1
    return pl.pallas_call(
        _edge_mm_body,
        grid=(_EP // _EB,),
        in_specs=[
            pl.BlockSpec((_EB, _D), lambda i: (jnp.minimum(i, nbe), 0)),
            pl.BlockSpec((_D, _D), lambda i: (0, 0)),
            pl.BlockSpec((1, _D), lambda i: (0, 0)),
        ],
        out_specs=pl.BlockSpec((_EB, _D), lambda i: (i, 0)),
        out_shape=jax.ShapeDtypeStruct((_EP, _D), jnp.float32),
    )(e, w, b.reshape(1, _D))


_MESH = plsc.VectorSubcoreMesh(core_axis_name="c", subcore_axis_name="s",
                               num_cores=2, num_subcores=_NSUB)
_SC_PARAMS = pltpu.CompilerParams(use_tc_tiling_on_sc=False)


def _zero_buf(zbuf):
    def zrow(i, carry):
        r = i // (_D // 16)
        col = (i % (_D // 16)) * 16
        zbuf[r, pl.ds(col, 16)] = jnp.zeros((16,), jnp.float32)
        return carry

    lax.fori_loop(0, (_SB * _D) // 16, zrow, 0)


def _zero_acc_stripe(acc, zbuf, sbase):
    for j in range(_STRIPE // _SB):
        pltpu.sync_copy(zbuf, acc.at[pl.ds(sbase + j * _SB, _SB)])


def _sigmoid16(g):
    return 1.0 / (1.0 + jnp.exp(-g))


def _make_sc_msg_pass():
    out_type = [jax.ShapeDtypeStruct((_EP, _D), jnp.float32),
                jax.ShapeDtypeStruct((2, _NA, _D), jnp.float32)]

    def buf():
        return pltpu.VMEM((_BLK, _D), jnp.float32)

    scratch = (
        [pltpu.VMEM((_NB // _SBB, _SB), jnp.int32)] * 2
        + [buf()] * 10
        + [pltpu.VMEM((_SB, _D), jnp.float32)]
        + [pltpu.VMEM_SHARED((_NA, _D), jnp.float32)]
        + [pltpu.SemaphoreType.DMA] * 4
    )

    def body(iu_hbm, iv_hbm, t2, t3, ta, b1e, gate, ph, iu_v, iv_v, *rest):
        sets = (rest[0:5], rest[5:10])
        mo_big = rest[10]
        acc = rest[11]
        gsems = (rest[12], rest[13])
        ssems = (rest[14], rest[15])

        c = lax.axis_index("c")
        s = lax.axis_index("s")
        wid = c * _NSUB + s
        base_e = wid * _EPT
        sbase = s * _STRIPE

        pltpu.sync_copy(iu_hbm.at[wid], iu_v)
        pltpu.sync_copy(iv_hbm.at[wid], iv_v)
        _zero_buf(mo_big)
        _zero_acc_stripe(acc, mo_big, sbase)
        plsc.subcore_barrier()

        def islice(slab, k):
            return slab.at[k // _SBB, pl.ds((k % _SBB) * _BLK, _BLK)]

        def issue_g(k, p):
            bu, bv, ba, b1 = sets[p][0:4]
            sem = gsems[p]
            return (pltpu.async_copy(t2.at[islice(iu_v, k)], bu, sem),
                    pltpu.async_copy(t3.at[islice(iv_v, k)], bv, sem),
                    pltpu.async_copy(ta.at[islice(iu_v, k)], ba, sem),
                    pltpu.async_copy(b1e.at[pl.ds(base_e + k * _BLK, _BLK)],
                                     b1, sem))

        def issue_gate(k, p):
            go = sets[p][4]
            return (pltpu.async_copy(go,
                                     gate.at[pl.ds(base_e + k * _BLK, _BLK)],
                                     ssems[p]),)

        def compute(p, rbase):
            bu, bv, ba, b1, go = sets[p]
            sls = [pl.ds(i * 16, 16) for i in range(_D // 16)]

            def rows(r2, cr):
                rr = [2 * r2, 2 * r2 + 1]
                pos = [(r, sl) for r in rr for sl in sls]
                gs = [b1[r, sl] + bu[r, sl] + bv[r, sl] for r, sl in pos]
                es = [jnp.exp(-g) for g in gs]
                sg = [1.0 / (1.0 + e2) for e2 in es]
                for i, (r, sl) in enumerate(pos):
                    go[r, sl] = gs[i]
                for i, (r, sl) in enumerate(pos):
                    mo_big[rbase + r, sl] = sg[i] * ba[r, sl]
                return cr

            lax.fori_loop(0, _BLK // 2, rows, 0)

        def chunk(j, carry):
            base = j * _CHK
            gd = {0: issue_g(base, 0), 1: issue_g(base + 1, 1)}
            sd = {}
            for bi in range(_CHK):
                p = bi % 2
                for d in gd.pop(bi):
                    d.wait()
                if bi >= 2:
                    for d in sd.pop(bi - 2):
                        d.wait()
                compute(p, (bi % _SBB) * _BLK)
                sd[bi] = issue_gate(base + bi, p)
                if bi % _SBB == _SBB - 1:
                    pltpu.sync_copy(
                        mo_big, acc.at[iv_v.at[(base + bi) // _SBB]],
                        add=True)
                if bi + 2 < _CHK:
                    gd[bi + 2] = issue_g(base + bi + 2, p)
            for d in sd.pop(_CHK - 2):
                d.wait()
            for d in sd.pop(_CHK - 1):
                d.wait()
            return carry

        lax.fori_loop(0, _NB // _CHK, chunk, 0)

        plsc.subcore_barrier()
        pltpu.sync_copy(acc.at[pl.ds(sbase, _STRIPE)],
                        ph.at[c, pl.ds(sbase, _STRIPE)])

    return pl.kernel(body, out_type=out_type, mesh=_MESH,
                     scratch_types=scratch, compiler_params=_SC_PARAMS)


def _make_sc_sig_pass():
    out_type = [jax.ShapeDtypeStruct((2, _NA, _D), jnp.float32)] * 2

    def buf():
        return pltpu.VMEM((_BLK, _D), jnp.float32)

    scratch = (
        [pltpu.VMEM((_NB // _SBB, _SB), jnp.int32)]
        + [buf()] * 2
        + [pltpu.VMEM((_SB, _D), jnp.float32)]
        + [pltpu.VMEM_SHARED((_NA, _D), jnp.float32)]
        + [pltpu.SemaphoreType.DMA] * 2
    )

    def body(dst_hbm, src_hbm, gate_f, gate_b, ps_f, ps_b, iv_v, *rest):
        sets = (rest[0], rest[1])
        so_big = rest[2]
        acc = rest[3]
        gsems = (rest[4], rest[5])

        c = lax.axis_index("c")
        s = lax.axis_index("s")
        wid = c * _NSUB + s
        base_e = wid * _EPT
        sbase = s * _STRIPE

        for idx_hbm, gate, ps in ((dst_hbm, gate_f, ps_f),
                                  (src_hbm, gate_b, ps_b)):
            pltpu.sync_copy(idx_hbm.at[wid], iv_v)
            _zero_buf(so_big)
            _zero_acc_stripe(acc, so_big, sbase)
            plsc.subcore_barrier()

            def issue_g(k, p):
                return (pltpu.async_copy(
                    gate.at[pl.ds(base_e + k * _BLK, _BLK)],
                    sets[p], gsems[p]),)

            def compute(p, rbase):
                bg = sets[p]
                sls = [pl.ds(i * 16, 16) for i in range(_D // 16)]

                def rows(r2, cr):
                    rr = [2 * r2, 2 * r2 + 1]
                    pos = [(r, sl) for r in rr for sl in sls]
                    es = [jnp.exp(-bg[r, sl]) for r, sl in pos]
                    sg = [1.0 / (1.0 + e2) for e2 in es]
                    for i, (r, sl) in enumerate(pos):
                        so_big[rbase + r, sl] = sg[i]
                    return cr

                lax.fori_loop(0, _BLK // 2, rows, 0)

            def chunk(j, carry):
                base = j * _CHK
                gd = {0: issue_g(base, 0), 1: issue_g(base + 1, 1)}
                for bi in range(_CHK):
                    p = bi % 2
                    for d in gd.pop(bi):
                        d.wait()
                    compute(p, (bi % _SBB) * _BLK)
                    if bi % _SBB == _SBB - 1:
                        pltpu.sync_copy(
                            so_big, acc.at[iv_v.at[(base + bi) // _SBB]],
                            add=True)
                    if bi + 2 < _CHK:
                        gd[bi + 2] = issue_g(base + bi + 2, p)
                return carry

            lax.fori_loop(0, _NB // _CHK, chunk, 0)

            plsc.subcore_barrier()
            pltpu.sync_copy(acc.at[pl.ds(sbase, _STRIPE)],
                            ps.at[c, pl.ds(sbase, _STRIPE)])
            plsc.subcore_barrier()

    return pl.kernel(body, out_type=out_type, mesh=_MESH,
                     scratch_types=scratch, compiler_params=_SC_PARAMS)


_sc_msg_pass = _make_sc_msg_pass()
_sc_sig_pass = _make_sc_sig_pass()


def _hfin_body(hin, a1h, fh, fs, bh, bs, g, b, out, stats):
    p = pl.program_id(0)
    i = pl.program_id(1)

    def comb(x):
        return jnp.sum(x[...], axis=0)

    hn = (a1h[...] + comb(fh) / (comb(fs) + _EPS_DEN)
          + comb(bh) / (comb(bs) + _EPS_DEN))

    @pl.when(jnp.logical_and(p == 0, i == 0))
    def _():
        stats[...] = jnp.zeros_like(stats)

    @pl.when(p == 0)
    def _():
        stats[0:1, :] = stats[0:1, :] + jnp.sum(hn, axis=0, keepdims=True)
        stats[1:2, :] = stats[1:2, :] + jnp.sum(hn * hn, axis=0, keepdims=True)
        out[...] = hn

    @pl.when(p == 1)
    def _():
        mu = stats[0:1, :] / _N
        var = stats[1:2, :] / _N - mu * mu
        xb = (hn - mu) * lax.rsqrt(var + _EPS_BN) * g[...] + b[...]
        out[...] = hin[...] + jnp.maximum(xb, 0.0)


def _h_finalize(hin, a1h, fh, fs, bh, bs, g, b):
    row = pl.BlockSpec((_RB, _D), lambda p, i: (i, 0))
    part = pl.BlockSpec((2, _RB, _D), lambda p, i: (0, i, 0))
    vec = pl.BlockSpec((1, _D), lambda p, i: (0, 0))
    return pl.pallas_call(
        _hfin_body,
        grid=(2, _N // _RB),
        in_specs=[row, row] + [part] * 4 + [vec, vec],
        out_specs=row,
        out_shape=jax.ShapeDtypeStruct((_N, _D), jnp.float32),
        scratch_shapes=[pltpu.VMEM((8, _D), jnp.float32)],
    )(hin, a1h, fh, fs, bh, bs, g.reshape(1, _D), b.reshape(1, _D))


def _efin_body(ein, gref, g, b, out, stats):
    p = pl.program_id(0)
    i = pl.program_id(1)
    ge = gref[...]

    @pl.when(jnp.logical_and(p == 0, i == 0))
    def _():
        stats[...] = jnp.zeros_like(stats)

    @pl.when(p == 0)
    def _():
        stats[0:1, :] = stats[0:1, :] + jnp.sum(ge, axis=0, keepdims=True)
        stats[1:2, :] = stats[1:2, :] + jnp.sum(ge * ge, axis=0, keepdims=True)
        out[...] = ge

    @pl.when(p == 1)
    def _():
        mu = stats[0:1, :] / _E
        var = stats[1:2, :] / _E - mu * mu
        xb = (ge - mu) * lax.rsqrt(var + _EPS_BN) * g[...] + b[...]
        out[...] = ein[...] + jnp.maximum(xb, 0.0)


def _e_finalize(ein, gate, g, b):
    row = pl.BlockSpec((_EB, _D), lambda p, i: (i, 0))
    vec = pl.BlockSpec((1, _D), lambda p, i: (0, 0))
    return pl.pallas_call(
        _efin_body,
        grid=(2, _E // _EB),
        in_specs=[row, row, vec, vec],
        out_specs=row,
        out_shape=jax.ShapeDtypeStruct((_E, _D), jnp.float32),
        scratch_shapes=[pltpu.VMEM((8, _D), jnp.float32)],
    )(ein, gate, g.reshape(1, _D), b.reshape(1, _D))


def _layer(h, e, src3, dst3, A1w, A1b, A2w, A2b, A3w, A3b,
           B1w, B1b, B2w, B2b, B3w, B3b, gh, bh, ge, be):
    a1h, a2h, a3h, b2h, b3h = _node_mm(
        h, (A1w, A2w, A3w, B2w, B3w), (A1b, A2b, A3b, B2b, B3b))
    b1e = _edge_mm(e, B1w, B1b)

    gate_f, fh = _sc_msg_pass(src3, dst3, b2h, b3h, a2h, b1e)
    gate_b, bhp = _sc_msg_pass(dst3, src3, b2h, b3h, a3h, b1e)
    fs, bs = _sc_sig_pass(dst3, src3, gate_f, gate_b)

    h_out = _h_finalize(h, a1h, fh, fs, bhp, bs, gh, bh)
    e_out = _e_finalize(e, gate_f, ge, be)
    return h_out, e_out


def kernel(h, e, edge_index, A1_w, A1_b, A2_w, A2_b, A3_w, A3_b,
           B1_w, B1_b, B2_w, B2_b, B3_w, B3_b, bn_h_g, bn_h_b,
           bn_e_g, bn_e_b):
    pad = jnp.full((_EP - _E,), _N, dtype=jnp.int32)
    src3 = jnp.concatenate([edge_index[0], pad]).reshape(_NW, _NB // _SBB, _SB)
    dst3 = jnp.concatenate([edge_index[1], pad]).reshape(_NW, _NB // _SBB, _SB)
    for i in range(_L):
        h, e = _layer(h, e, src3, dst3,
                      A1_w[i], A1_b[i], A2_w[i], A2_b[i], A3_w[i], A3_b[i],
                      B1_w[i], B1_b[i], B2_w[i], B2_b[i], B3_w[i], B3_b[i],
                      bn_h_g[i], bn_h_b[i], bn_e_g[i], bn_e_b[i])
    return (h, e)
